# Initial kernel scaffold; baseline (speedup 1.0000x reference)
import jax
import jax.numpy as jnp
from jax.experimental import pallas as pl


def kernel(imgs, labels, q_params, op_embs, num_transforms_embs, scale_embs):
    raise NotImplementedError("write your pallas kernel here")


# speedup vs baseline: 5.8773x; 5.8773x over previous
import functools

import jax
import jax.numpy as jnp
from jax import lax
from jax.experimental import pallas as pl
from jax.experimental.pallas import tpu as pltpu
from jax.experimental.pallas import tpu_sc as plsc


def _gather_q(q_params, labels):
    B = labels.shape[0]
    H = q_params.shape[1]
    info = plsc.get_sparse_core_info()
    nw = info.num_cores * info.num_subcores
    b_per_w = B // nw
    mesh = plsc.VectorSubcoreMesh(core_axis_name="c", subcore_axis_name="s")

    @functools.partial(
        pl.kernel,
        mesh=mesh,
        out_type=jax.ShapeDtypeStruct((B, H), jnp.float32),
        scratch_types=[
            pltpu.VMEM((b_per_w,), jnp.int32),
            pltpu.VMEM((b_per_w, H), jnp.float32),
            pltpu.SemaphoreType.DMA,
        ],
    )
    def k(table_hbm, idx_hbm, out_hbm, idx_v, rows_v, sem):
        wid = lax.axis_index("s") * info.num_cores + lax.axis_index("c")
        base = wid * b_per_w
        pltpu.sync_copy(idx_hbm.at[pl.ds(base, b_per_w)], idx_v)
        pltpu.async_copy(table_hbm.at[idx_v], rows_v, sem).wait()
        pltpu.sync_copy(rows_v, out_hbm.at[pl.ds(base, b_per_w)])

    return k(q_params, labels)


def _sample_body(num_ops, num_scales, max_t, bm, cols,
                 q_ref, w_ref, g3_ref, g2_ref, aug_ref, augout_ref, scout_ref):
    acc = jnp.dot(q_ref[...], w_ref[...], preferred_element_type=jnp.float32)
    win = num_ops * num_scales
    col = lax.broadcasted_iota(jnp.int32, (bm, cols), 1)
    opcol = jnp.where(col < win, col // num_scales, num_ops + 1)

    a0 = acc[:, win:win + 1] + g2_ref[:, 0:1]
    a1 = acc[:, win + 1:win + 2] + g2_ref[:, 1:2]
    a2 = acc[:, win + 2:win + 3] + g2_ref[:, 2:3]
    nt_idx = jnp.where(a1 > a0, 1, 0)
    nt_idx = jnp.where(a2 > jnp.maximum(a0, a1), 2, nt_idx)
    n_transforms = nt_idx + 1

    col128 = lax.broadcasted_iota(jnp.int32, (bm, 128), 1)
    augout = jnp.zeros((bm, 128), jnp.int32)
    scout = jnp.zeros((bm, 128), jnp.int32)
    pad = jnp.zeros((bm, cols - win), jnp.float32)
    for t in range(max_t):
        noise_t = g3_ref[:, t * 128:t * 128 + num_scales]
        tiled = jnp.concatenate([noise_t] * num_ops + [pad], axis=1)
        ind_t = aug_ref[:, t:t + 1]
        masked = jnp.where(opcol == ind_t, acc + tiled, -1e30)
        mx = jnp.max(masked, axis=1, keepdims=True)
        firstcol = jnp.min(jnp.where(masked == mx, col, cols), axis=1,
                           keepdims=True)
        scale_t = firstcol - ind_t * num_scales
        aug_t = jnp.where(t < n_transforms, ind_t, 0)
        augout = jnp.where(col128 == t, aug_t, augout)
        scout = jnp.where(col128 == t, scale_t, scout)
    augout_ref[...] = augout
    scout_ref[...] = scout


def kernel(imgs, labels, q_params, op_embs, num_transforms_embs, scale_embs):
    B = imgs.shape[0]
    num_ops, num_scales, H = scale_embs.shape
    max_t = num_transforms_embs.shape[0]
    win = num_ops * num_scales
    cols = ((win + max_t) + 127) // 128 * 128

    key = jax.random.key(42)
    k1, k2, k3 = jax.random.split(key, 3)
    aug_raw = jax.random.randint(k1, (B, max_t), 0, num_ops)
    g2 = jax.random.gumbel(k2, (B, max_t), jnp.float32)
    g3 = jax.random.gumbel(k3, (B * max_t, num_scales), jnp.float32)

    w = jnp.concatenate(
        [scale_embs.reshape(win, H), num_transforms_embs,
         jnp.zeros((cols - win - max_t, H), jnp.float32)], axis=0).T
    g3p = jnp.pad(g3.reshape(B, max_t, num_scales),
                  ((0, 0), (0, 0), (0, 128 - num_scales))).reshape(B, max_t * 128)
    g2p = jnp.pad(g2, ((0, 0), (0, 128 - max_t)))
    augp = jnp.pad(aug_raw.astype(jnp.int32), ((0, 0), (0, 128 - max_t)))

    q = _gather_q(q_params, labels.astype(jnp.int32))

    bm = 512
    grid = (B // bm,)
    augout, scout = pl.pallas_call(
        functools.partial(_sample_body, num_ops, num_scales, max_t, bm, cols),
        grid=grid,
        in_specs=[
            pl.BlockSpec((bm, H), lambda i: (i, 0)),
            pl.BlockSpec((H, cols), lambda i: (0, 0)),
            pl.BlockSpec((bm, max_t * 128), lambda i: (i, 0)),
            pl.BlockSpec((bm, 128), lambda i: (i, 0)),
            pl.BlockSpec((bm, 128), lambda i: (i, 0)),
        ],
        out_specs=[
            pl.BlockSpec((bm, 128), lambda i: (i, 0)),
            pl.BlockSpec((bm, 128), lambda i: (i, 0)),
        ],
        out_shape=[
            jax.ShapeDtypeStruct((B, 128), jnp.int32),
            jax.ShapeDtypeStruct((B, 128), jnp.int32),
        ],
    )(q, w, g3p, g2p, augp)

    return (augout[:, :max_t], scout[:, :max_t])


# speedup vs baseline: 10.3934x; 1.7684x over previous
import functools

import numpy as np
import jax
import jax.numpy as jnp
from jax import lax
from jax.experimental import pallas as pl
from jax.experimental.pallas import tpu as pltpu
from jax.experimental.pallas import tpu_sc as plsc


def _np_rotl(x, d):
    d = np.uint32(d)
    return (x << d) | (x >> np.uint32(32 - d))


def _np_threefry2x32(k0, k1, c1, c2):
    rot = ((13, 15, 26, 6), (17, 29, 16, 24))
    ks0 = np.uint32(k0)
    ks1 = np.uint32(k1)
    ks2 = ks0 ^ ks1 ^ np.uint32(0x1BD11BDA)
    ks = (ks0, ks1, ks2)
    x0 = (c1 + ks0).astype(np.uint32)
    x1 = (c2 + ks1).astype(np.uint32)
    for i in range(5):
        for r in rot[i % 2]:
            x0 = (x0 + x1).astype(np.uint32)
            x1 = _np_rotl(x1, r)
            x1 = x0 ^ x1
        x0 = (x0 + ks[(i + 1) % 3]).astype(np.uint32)
        x1 = (x1 + ks[(i + 2) % 3] + np.uint32(i + 1)).astype(np.uint32)
    return x0, x1


def _np_split(key, num):
    idx = np.arange(num, dtype=np.uint64)
    c1 = (idx >> np.uint64(32)).astype(np.uint32)
    c2 = (idx & np.uint64(0xFFFFFFFF)).astype(np.uint32)
    b1, b2 = _np_threefry2x32(key[0], key[1], c1, c2)
    return [(b1[i], b2[i]) for i in range(num)]

_KEY = (np.uint32(0), np.uint32(42))
_K1, _K2, _K3 = _np_split(_KEY, 3)
_KA = _np_split(_K1, 2)[1]


def _rotl(x, d):
    return lax.shift_left(x, np.uint32(d)) | lax.shift_right_logical(
        x, np.uint32(32 - d))


def _threefry(ks0, ks1, c2):
    rot = ((13, 15, 26, 6), (17, 29, 16, 24))
    ks2 = ks0 ^ ks1 ^ np.uint32(0x1BD11BDA)
    ks = (ks0, ks1, ks2)
    x0 = ks0 + jnp.zeros_like(c2)
    x1 = c2 + ks1
    for i in range(5):
        for r in rot[i % 2]:
            x0 = x0 + x1
            x1 = _rotl(x1, r)
            x1 = x0 ^ x1
        x0 = x0 + ks[(i + 1) % 3]
        x1 = x1 + ks[(i + 2) % 3] + np.uint32(i + 1)
    return x0 ^ x1


def _gather_q(q_params, labels):
    B = labels.shape[0]
    H = q_params.shape[1]
    info = plsc.get_sparse_core_info()
    nw = info.num_cores * info.num_subcores
    b_per_w = B // nw
    mesh = plsc.VectorSubcoreMesh(core_axis_name="c", subcore_axis_name="s")

    @functools.partial(
        pl.kernel,
        mesh=mesh,
        out_type=jax.ShapeDtypeStruct((B, H), jnp.float32),
        scratch_types=[
            pltpu.VMEM((b_per_w,), jnp.int32),
            pltpu.VMEM((b_per_w, H), jnp.float32),
            pltpu.SemaphoreType.DMA,
        ],
    )
    def k(table_hbm, idx_hbm, out_hbm, idx_v, rows_v, sem):
        wid = lax.axis_index("s") * info.num_cores + lax.axis_index("c")
        base = wid * b_per_w
        pltpu.sync_copy(idx_hbm.at[pl.ds(base, b_per_w)], idx_v)
        pltpu.async_copy(table_hbm.at[idx_v], rows_v, sem).wait()
        pltpu.sync_copy(rows_v, out_hbm.at[pl.ds(base, b_per_w)])

    return k(q_params, labels)


def _sample_body(num_ops, num_scales, max_t, bm, cols,
                 q_ref, w_ref, augout_ref, scout_ref):
    win = num_ops * num_scales
    nts = max_t * num_scales

    i = pl.program_id(0)
    rowg = lax.broadcasted_iota(jnp.uint32, (bm, 128), 0) + np.uint32(bm) * i.astype(jnp.uint32)
    col = lax.broadcasted_iota(jnp.uint32, (bm, 128), 1)
    is_g3 = col < np.uint32(nts)
    is_g2 = col < np.uint32(nts + max_t)
    counts = jnp.where(
        is_g3, np.uint32(nts) * rowg + col,
        jnp.where(is_g2, np.uint32(max_t) * rowg + col - np.uint32(nts),
                  np.uint32(max_t) * rowg + col - np.uint32(nts + max_t)))
    k0 = jnp.where(is_g3, np.uint32(_K3[0]),
                   jnp.where(is_g2, np.uint32(_K2[0]), np.uint32(_KA[0])))
    k1 = jnp.where(is_g3, np.uint32(_K3[1]),
                   jnp.where(is_g2, np.uint32(_K2[1]), np.uint32(_KA[1])))
    bits = _threefry(k0, k1, counts)

    fb = lax.shift_right_logical(bits, np.uint32(9)) | np.uint32(0x3F800000)
    f = lax.bitcast_convert_type(fb, jnp.float32) - 1.0
    tiny = np.float32(np.finfo(np.float32).tiny)
    u = jnp.maximum(tiny, f * np.float32(1.0) + tiny)
    gum = -jnp.log(-jnp.log(u))
    augbits = lax.bitcast_convert_type(bits & np.uint32(15), jnp.int32)

    acc = jnp.dot(q_ref[...], w_ref[...], preferred_element_type=jnp.float32)
    colc = lax.broadcasted_iota(jnp.int32, (bm, cols), 1)
    opcol = jnp.where(colc < win, colc // num_scales, num_ops + 1)

    a0 = acc[:, win:win + 1] + gum[:, nts:nts + 1]
    a1 = acc[:, win + 1:win + 2] + gum[:, nts + 1:nts + 2]
    a2 = acc[:, win + 2:win + 3] + gum[:, nts + 2:nts + 3]
    nt_idx = jnp.where(a1 > a0, 1, 0)
    nt_idx = jnp.where(a2 > jnp.maximum(a0, a1), 2, nt_idx)
    n_transforms = nt_idx + 1

    pad = jnp.zeros((bm, cols - win), jnp.float32)
    aug_cols = []
    sc_cols = []
    for t in range(max_t):
        noise_t = gum[:, t * num_scales:(t + 1) * num_scales]
        tiled = jnp.concatenate([noise_t] * num_ops + [pad], axis=1)
        ind_t = augbits[:, nts + max_t + t:nts + max_t + t + 1]
        masked = jnp.where(opcol == ind_t, acc + tiled, -1e30)
        mx = jnp.max(masked, axis=1, keepdims=True)
        firstcol = jnp.min(jnp.where(masked == mx, colc, cols), axis=1,
                           keepdims=True)
        sc_cols.append(firstcol - ind_t * num_scales)
        aug_cols.append(jnp.where(t < n_transforms, ind_t, 0))
    augout_ref[...] = jnp.concatenate(aug_cols, axis=1)
    scout_ref[...] = jnp.concatenate(sc_cols, axis=1)


def kernel(imgs, labels, q_params, op_embs, num_transforms_embs, scale_embs):
    B = imgs.shape[0]
    num_ops, num_scales, H = scale_embs.shape
    max_t = num_transforms_embs.shape[0]
    win = num_ops * num_scales
    cols = ((win + max_t) + 127) // 128 * 128

    w = jnp.concatenate(
        [scale_embs.reshape(win, H), num_transforms_embs,
         jnp.zeros((cols - win - max_t, H), jnp.float32)], axis=0).T

    q = _gather_q(q_params, labels.astype(jnp.int32))

    bm = 512
    grid = (B // bm,)
    augout, scout = pl.pallas_call(
        functools.partial(_sample_body, num_ops, num_scales, max_t, bm, cols),
        grid=grid,
        in_specs=[
            pl.BlockSpec((bm, H), lambda i: (i, 0)),
            pl.BlockSpec((H, cols), lambda i: (0, 0)),
        ],
        out_specs=[
            pl.BlockSpec((bm, max_t), lambda i: (i, 0)),
            pl.BlockSpec((bm, max_t), lambda i: (i, 0)),
        ],
        out_shape=[
            jax.ShapeDtypeStruct((B, max_t), jnp.int32),
            jax.ShapeDtypeStruct((B, max_t), jnp.int32),
        ],
    )(q, w)

    return (augout, scout)
